# baseline (speedup 1.0000x reference)
import functools

import jax
import jax.numpy as jnp
from jax import lax
from jax.experimental import pallas as pl
from jax.experimental.pallas import tpu as pltpu
import jax.experimental.pallas.tpu_sc as plsc

K = 16
KP1 = 17


_TOPK_ROWS = 256


def _topk_body(pts_ref, ptile_ref, out_ref):
    b = pl.program_id(0)
    t = pl.program_id(1)
    pT = pts_ref[0]
    ptile = ptile_ref[0]
    n = pT.shape[1]
    r = ptile.shape[1]

    ones = jnp.ones((3, 1), dtype=jnp.float32)
    dn = (((0,), (0,)), ((), ()))
    prec = lax.Precision.HIGHEST
    r_tile = lax.dot_general(ptile * ptile, ones, dn, precision=prec)
    r_all = lax.dot_general(ones, pT * pT, dn, precision=prec)
    cross = lax.dot_general(ptile, pT, dn, precision=lax.Precision.DEFAULT)
    d = jnp.abs(r_tile - 2.0 * cross + r_all)

    inf = jnp.float32(jnp.inf)
    iota = lax.broadcasted_iota(jnp.int32, (r, n), 1)
    cols = []
    for step in range(KP1):
        am = jnp.argmin(d, axis=1).astype(jnp.int32)[:, None]
        if step > 0:
            cols.append(am)
        d = jnp.where(iota == am, inf, d)
    out_ref[0] = jnp.concatenate(cols, axis=1) + b * n


def _topk_call(pts_t):
    bsz, _, n = pts_t.shape
    grid = (bsz, n // _TOPK_ROWS)
    return pl.pallas_call(
        _topk_body,
        grid=grid,
        in_specs=[
            pl.BlockSpec((1, 3, n), lambda b, t: (b, 0, 0)),
            pl.BlockSpec((1, 3, _TOPK_ROWS), lambda b, t: (b, 0, t)),
        ],
        out_specs=pl.BlockSpec((1, _TOPK_ROWS, K), lambda b, t: (b, t, 0)),
        out_shape=jax.ShapeDtypeStruct((bsz, n, K), jnp.int32),
    )(pts_t, pts_t)


_NC, _NS = 2, 16
_NW = _NC * _NS
_CH = 128


def _gather_body(table_hbm, idx_hbm, out_hbm, idx_v, rows_v, sem):
    wid = lax.axis_index("s") * _NC + lax.axis_index("c")
    nch = idx_v.shape[0]
    pltpu.sync_copy(idx_hbm.at[wid], idx_v)

    def chunk(j, carry):
        pltpu.async_copy(table_hbm.at[idx_v.at[j]], rows_v, sem).wait()
        pltpu.sync_copy(rows_v, out_hbm.at[wid, j])
        return carry

    lax.fori_loop(0, nch, chunk, 0)


def _gather_call(table, gidx):
    tot = gidx.shape[0]
    d = table.shape[1]
    nch = tot // (_NW * _CH)
    mesh = plsc.VectorSubcoreMesh(core_axis_name="c", subcore_axis_name="s")
    f = pl.kernel(
        _gather_body,
        out_type=jax.ShapeDtypeStruct((_NW, nch, _CH, d), table.dtype),
        mesh=mesh,
        scratch_types=[
            pltpu.VMEM((nch, _CH), jnp.int32),
            pltpu.VMEM((_CH, d), table.dtype),
            pltpu.SemaphoreType.DMA,
        ],
    )
    out = f(table, gidx.reshape(_NW, nch, _CH))
    return out.reshape(tot, d)


_MLP_ROWS = 128


def _mlp_body(knnf_ref, feats_ref, w1p_ref, w1d_ref, b1_ref, w2_ref, b2_ref,
              out_ref):
    r = feats_ref.shape[0]
    dproj = out_ref.shape[1]
    prec = lax.Precision.DEFAULT
    c = jnp.dot(feats_ref[...], w1d_ref[...], precision=prec,
                preferred_element_type=jnp.float32) + b1_ref[...]
    h = jnp.dot(knnf_ref[...], w1p_ref[...],
                precision=prec, preferred_element_type=jnp.float32)
    h = h.reshape(r, K, h.shape[1]) + c[:, None, :]
    h = h.reshape(r * K, h.shape[2])
    h1 = _gelu(h)
    h2 = _gelu(jnp.dot(h1, w2_ref[...], precision=prec,
                       preferred_element_type=jnp.float32) + b2_ref[...])
    out_ref[...] = jnp.sum(h2.reshape(r, K, dproj), axis=1) * (1.0 / K)


def _gelu(x):
    return 0.5 * x * (1.0 + lax.erf(x * 0.7071067811865476))


def _mlp_call(knnf, feats, w1p, w1d, b1, w2, b2):
    p, dfeat = feats.shape
    grid = (p // _MLP_ROWS,)
    return pl.pallas_call(
        _mlp_body,
        grid=grid,
        in_specs=[
            pl.BlockSpec((_MLP_ROWS * K, dfeat), lambda s: (s, 0)),
            pl.BlockSpec((_MLP_ROWS, dfeat), lambda s: (s, 0)),
            pl.BlockSpec(w1p.shape, lambda s: (0, 0)),
            pl.BlockSpec(w1d.shape, lambda s: (0, 0)),
            pl.BlockSpec((1, b1.shape[1]), lambda s: (0, 0)),
            pl.BlockSpec(w2.shape, lambda s: (0, 0)),
            pl.BlockSpec((1, b2.shape[1]), lambda s: (0, 0)),
        ],
        out_specs=pl.BlockSpec((_MLP_ROWS, 128), lambda s: (s, 0)),
        out_shape=jax.ShapeDtypeStruct((p, 128), jnp.float32),
    )(knnf, feats, w1p, w1d, b1, w2, b2)


def kernel(points, features, W1, b1, W2, b2):
    bsz, n, _ = points.shape
    dfeat = features.shape[-1]
    pts_t = jnp.transpose(points, (0, 2, 1))
    b1r = b1.reshape(1, -1)
    b2r = b2.reshape(1, -1)
    w1p = W1[0:128]
    w1d = W1[128:256] - w1p
    h = bsz // 2
    outs = []
    for i in range(2):
        pts_h = pts_t[i * h:(i + 1) * h]
        feats_h = features[i * h:(i + 1) * h].reshape(h * n, dfeat)
        gidx = _topk_call(pts_h)
        knnf = _gather_call(feats_h, gidx.reshape(-1))
        outs.append(_mlp_call(knnf, feats_h, w1p, w1d, b1r, W2, b2r))
    return jnp.concatenate(outs, axis=0).reshape(bsz, n, dfeat)


# speedup vs baseline: 1.0521x; 1.0521x over previous
import functools

import jax
import jax.numpy as jnp
from jax import lax
from jax.experimental import pallas as pl
from jax.experimental.pallas import tpu as pltpu
import jax.experimental.pallas.tpu_sc as plsc

K = 16
KP1 = 17


_TOPK_ROWS = 512


def _topk_body(pts_ref, ptile_ref, out_ref):
    b = pl.program_id(0)
    t = pl.program_id(1)
    pT = pts_ref[0]
    ptile = ptile_ref[0]
    n = pT.shape[1]
    r = ptile.shape[1]

    ones = jnp.ones((3, 1), dtype=jnp.float32)
    dn = (((0,), (0,)), ((), ()))
    prec = lax.Precision.HIGHEST
    r_tile = lax.dot_general(ptile * ptile, ones, dn, precision=prec)
    r_all = lax.dot_general(ones, pT * pT, dn, precision=prec)
    cross = lax.dot_general(ptile, pT, dn, precision=lax.Precision.DEFAULT)
    d = jnp.abs(r_tile - 2.0 * cross + r_all)

    inf = jnp.float32(jnp.inf)
    iota = lax.broadcasted_iota(jnp.int32, (r, n), 1)
    cols = []
    for step in range(KP1):
        am = jnp.argmin(d, axis=1).astype(jnp.int32)[:, None]
        if step > 0:
            cols.append(am)
        d = jnp.where(iota == am, inf, d)
    out_ref[0] = jnp.concatenate(cols, axis=1) + b * n


def _topk_call(pts_t):
    bsz, _, n = pts_t.shape
    grid = (bsz, n // _TOPK_ROWS)
    return pl.pallas_call(
        _topk_body,
        grid=grid,
        in_specs=[
            pl.BlockSpec((1, 3, n), lambda b, t: (b, 0, 0)),
            pl.BlockSpec((1, 3, _TOPK_ROWS), lambda b, t: (b, 0, t)),
        ],
        out_specs=pl.BlockSpec((1, _TOPK_ROWS, K), lambda b, t: (b, t, 0)),
        out_shape=jax.ShapeDtypeStruct((bsz, n, K), jnp.int32),
    )(pts_t, pts_t)


_NC, _NS = 2, 16
_NW = _NC * _NS
_CH = 128


def _gather_body(table_hbm, idx_hbm, out_hbm, idx_v, rows_v, sem):
    wid = lax.axis_index("s") * _NC + lax.axis_index("c")
    nch = idx_v.shape[0]
    pltpu.sync_copy(idx_hbm.at[wid], idx_v)

    def chunk(j, carry):
        pltpu.async_copy(table_hbm.at[idx_v.at[j]], rows_v, sem).wait()
        pltpu.sync_copy(rows_v, out_hbm.at[wid, j])
        return carry

    lax.fori_loop(0, nch, chunk, 0)


def _gather_call(table, gidx):
    tot = gidx.shape[0]
    d = table.shape[1]
    nch = tot // (_NW * _CH)
    mesh = plsc.VectorSubcoreMesh(core_axis_name="c", subcore_axis_name="s")
    f = pl.kernel(
        _gather_body,
        out_type=jax.ShapeDtypeStruct((_NW, nch, _CH, d), table.dtype),
        mesh=mesh,
        scratch_types=[
            pltpu.VMEM((nch, _CH), jnp.int32),
            pltpu.VMEM((_CH, d), table.dtype),
            pltpu.SemaphoreType.DMA,
        ],
    )
    out = f(table, gidx.reshape(_NW, nch, _CH))
    return out.reshape(tot, d)


_MLP_ROWS = 256


def _mlp_body(knnf_ref, feats_ref, w1p_ref, w1d_ref, b1_ref, w2_ref, b2_ref,
              out_ref):
    r = feats_ref.shape[0]
    dproj = out_ref.shape[1]
    prec = lax.Precision.DEFAULT
    c = jnp.dot(feats_ref[...], w1d_ref[...], precision=prec,
                preferred_element_type=jnp.float32) + b1_ref[...]
    h = jnp.dot(knnf_ref[...], w1p_ref[...],
                precision=prec, preferred_element_type=jnp.float32)
    h = h.reshape(r, K, h.shape[1]) + c[:, None, :]
    h = h.reshape(r * K, h.shape[2])
    h1 = _gelu(h)
    h2 = _gelu(jnp.dot(h1, w2_ref[...], precision=prec,
                       preferred_element_type=jnp.float32) + b2_ref[...])
    out_ref[...] = jnp.sum(h2.reshape(r, K, dproj), axis=1) * (1.0 / K)


def _gelu(x):
    return 0.5 * x * (1.0 + lax.erf(x * 0.7071067811865476))


def _mlp_call(knnf, feats, w1p, w1d, b1, w2, b2):
    p, dfeat = feats.shape
    grid = (p // _MLP_ROWS,)
    return pl.pallas_call(
        _mlp_body,
        grid=grid,
        in_specs=[
            pl.BlockSpec((_MLP_ROWS * K, dfeat), lambda s: (s, 0)),
            pl.BlockSpec((_MLP_ROWS, dfeat), lambda s: (s, 0)),
            pl.BlockSpec(w1p.shape, lambda s: (0, 0)),
            pl.BlockSpec(w1d.shape, lambda s: (0, 0)),
            pl.BlockSpec((1, b1.shape[1]), lambda s: (0, 0)),
            pl.BlockSpec(w2.shape, lambda s: (0, 0)),
            pl.BlockSpec((1, b2.shape[1]), lambda s: (0, 0)),
        ],
        out_specs=pl.BlockSpec((_MLP_ROWS, 128), lambda s: (s, 0)),
        out_shape=jax.ShapeDtypeStruct((p, 128), jnp.float32),
    )(knnf, feats, w1p, w1d, b1, w2, b2)


def kernel(points, features, W1, b1, W2, b2):
    bsz, n, _ = points.shape
    dfeat = features.shape[-1]
    pts_t = jnp.transpose(points, (0, 2, 1))
    b1r = b1.reshape(1, -1)
    b2r = b2.reshape(1, -1)
    w1p = W1[0:128]
    w1d = W1[128:256] - w1p
    h = bsz // 2
    outs = []
    for i in range(2):
        pts_h = pts_t[i * h:(i + 1) * h]
        feats_h = features[i * h:(i + 1) * h].reshape(h * n, dfeat)
        gidx = _topk_call(pts_h)
        knnf = _gather_call(feats_h, gidx.reshape(-1))
        outs.append(_mlp_call(knnf, feats_h, w1p, w1d, b1r, W2, b2r))
    return jnp.concatenate(outs, axis=0).reshape(bsz, n, dfeat)
